# baseline (speedup 1.0000x reference)
import jax
import jax.numpy as jnp
from jax import lax
from jax.experimental import pallas as pl
from jax.experimental.pallas import tpu as pltpu

B, S, D, V = 1, 2048, 1024, 50258
L, H, FF = 2, 16, 4096
NPEERS, TOPK = 64, 8
HD = D // H

BM = 256
BV = 2048
NJ = (V + BV - 1) // BV
NEG = -1e30
BF = jnp.bfloat16

_arb = lambda n: pltpu.CompilerParams(dimension_semantics=("arbitrary",) * n)


def _combine_body(pw_ref, af_ref, nz_ref, resp_ref, x_ref):
    pw = pw_ref[...]
    af = af_ref[...]
    nz = nz_ref[...]
    n = jnp.sum(af)
    mean = jnp.sum(pw * af) / n
    std = jnp.sqrt(jnp.sum(af * (pw - mean) ** 2) / jnp.maximum(n - 1.0, 1.0))
    scores = jnp.where(af > 0, pw + nz * (std + 1e-7), -1e9)
    ms = []
    s = scores
    for _ in range(TOPK):
        m = jnp.max(s)
        ms.append(m)
        s = jnp.where(s >= m, NEG, s)
    ps = [jnp.exp(m - ms[0]) for m in ms]
    z = ps[0]
    for p in ps[1:]:
        z = z + p
    acc = (ps[0] / z) * resp_ref[0]
    for e in range(1, TOPK):
        acc = acc + (ps[e] / z) * resp_ref[e]
    x_ref[...] = acc


def _combine(pw, af, nz, resp):
    return pl.pallas_call(
        _combine_body,
        grid=(S // BM,),
        in_specs=[
            pl.BlockSpec((1, NPEERS), lambda i: (0, 0)),
            pl.BlockSpec((1, NPEERS), lambda i: (0, 0)),
            pl.BlockSpec((1, NPEERS), lambda i: (0, 0)),
            pl.BlockSpec((TOPK, BM, D), lambda i: (0, i, 0)),
        ],
        out_specs=pl.BlockSpec((BM, D), lambda i: (i, 0)),
        out_shape=jax.ShapeDtypeStruct((S, D), jnp.float32),
        compiler_params=_arb(1),
    )(pw, af, nz, resp)


def _qkv_body(x_ref, w_ref, b_ref, o_ref):
    xb = x_ref[...].astype(BF)
    wb = w_ref[0].astype(BF)
    o_ref[...] = (
        jnp.dot(xb, wb, preferred_element_type=jnp.float32) + b_ref[0]
    )


def _qkv(x, wqkv, bqkv, l):
    return pl.pallas_call(
        _qkv_body,
        grid=(S // BM, 3),
        in_specs=[
            pl.BlockSpec((BM, D), lambda i, j: (i, 0)),
            pl.BlockSpec((1, D, D), lambda i, j: (l, 0, j)),
            pl.BlockSpec((1, 1, D), lambda i, j: (l, 0, j)),
        ],
        out_specs=pl.BlockSpec((BM, D), lambda i, j: (i, j)),
        out_shape=jax.ShapeDtypeStruct((S, 3 * D), jnp.float32),
        compiler_params=_arb(2),
    )(x, wqkv, bqkv)


def _attn_one(q, k, v):
    s = lax.dot_general(q.astype(BF), k.astype(BF), (((1,), (1,)), ((), ())),
                        preferred_element_type=jnp.float32)
    s = s * (1.0 / (HD ** 0.5))
    m = jnp.max(s, axis=1, keepdims=True)
    p = jnp.exp(s - m)
    l = jnp.sum(p, axis=1, keepdims=True)
    att = (p / l).astype(BF)
    return jnp.dot(att, v.astype(BF), preferred_element_type=jnp.float32)


def _attn_body(q_ref, k_ref, v_ref, o_ref):
    q = q_ref[...]
    k = k_ref[...]
    v = v_ref[...]
    o0 = _attn_one(q[:, :HD], k[:, :HD], v[:, :HD])
    o1 = _attn_one(q[:, HD:], k[:, HD:], v[:, HD:])
    o_ref[...] = jnp.concatenate([o0, o1], axis=1)


def _attn(qkv):
    hp = H // 2
    return pl.pallas_call(
        _attn_body,
        grid=(hp, S // BM),
        in_specs=[
            pl.BlockSpec((BM, 2 * HD), lambda h, i: (i, h)),
            pl.BlockSpec((S, 2 * HD), lambda h, i: (0, hp + h)),
            pl.BlockSpec((S, 2 * HD), lambda h, i: (0, 2 * hp + h)),
        ],
        out_specs=pl.BlockSpec((BM, 2 * HD), lambda h, i: (i, h)),
        out_shape=jax.ShapeDtypeStruct((S, D), jnp.float32),
        compiler_params=_arb(2),
    )(qkv, qkv, qkv)


def _layernorm(y, g, b):
    mu = jnp.mean(y, axis=1, keepdims=True)
    var = jnp.mean((y - mu) ** 2, axis=1, keepdims=True)
    return (y - mu) * lax.rsqrt(var + 1e-5) * g + b


def _oproj_body(o_ref, w_ref, b_ref, x_ref, g_ref, bb_ref, y_ref):
    y = x_ref[...] + jnp.dot(o_ref[...].astype(BF), w_ref[0].astype(BF),
                             preferred_element_type=jnp.float32) + b_ref[0]
    y_ref[...] = _layernorm(y, g_ref[0], bb_ref[0])


def _oproj_ln(o, wo, bo, x, g, bb, l):
    return pl.pallas_call(
        _oproj_body,
        grid=(S // BM,),
        in_specs=[
            pl.BlockSpec((BM, D), lambda i: (i, 0)),
            pl.BlockSpec((1, D, D), lambda i: (l, 0, 0)),
            pl.BlockSpec((1, 1, D), lambda i: (l, 0, 0)),
            pl.BlockSpec((BM, D), lambda i: (i, 0)),
            pl.BlockSpec((1, 1, D), lambda i: (l, 0, 0)),
            pl.BlockSpec((1, 1, D), lambda i: (l, 0, 0)),
        ],
        out_specs=pl.BlockSpec((BM, D), lambda i: (i, 0)),
        out_shape=jax.ShapeDtypeStruct((S, D), jnp.float32),
        compiler_params=_arb(1),
    )(o, wo, bo, x, g, bb)


FC = 1024


def _ffn_body(x_ref, w1_ref, b1_ref, w2_ref, b2_ref, g_ref, bb_ref, y_ref,
              acc_ref):
    c = pl.program_id(1)
    h = jnp.maximum(
        jnp.dot(x_ref[...].astype(BF), w1_ref[0].astype(BF),
                preferred_element_type=jnp.float32) + b1_ref[0], 0.0)
    part = jnp.dot(h.astype(BF), w2_ref[0].astype(BF),
                   preferred_element_type=jnp.float32)

    @pl.when(c == 0)
    def _():
        acc_ref[...] = x_ref[...] + b2_ref[0] + part

    @pl.when(c > 0)
    def _():
        acc_ref[...] = acc_ref[...] + part

    @pl.when(c == FF // FC - 1)
    def _():
        y_ref[...] = _layernorm(acc_ref[...], g_ref[0], bb_ref[0])


def _ffn_ln(x, w1, b1, w2, b2, g, bb, l):
    return pl.pallas_call(
        _ffn_body,
        grid=(S // BM, FF // FC),
        in_specs=[
            pl.BlockSpec((BM, D), lambda i, c: (i, 0)),
            pl.BlockSpec((1, D, FC), lambda i, c: (l, 0, c)),
            pl.BlockSpec((1, 1, FC), lambda i, c: (l, 0, c)),
            pl.BlockSpec((1, FC, D), lambda i, c: (l, c, 0)),
            pl.BlockSpec((1, 1, D), lambda i, c: (l, 0, 0)),
            pl.BlockSpec((1, 1, D), lambda i, c: (l, 0, 0)),
            pl.BlockSpec((1, 1, D), lambda i, c: (l, 0, 0)),
        ],
        out_specs=pl.BlockSpec((BM, D), lambda i, c: (i, 0)),
        out_shape=jax.ShapeDtypeStruct((S, D), jnp.float32),
        scratch_shapes=[pltpu.VMEM((BM, D), jnp.float32)],
        compiler_params=_arb(2),
    )(x, w1, b1, w2, b2, g, bb)


def _dec_body(x_ref, w_ref, lab_ref, logit_ref, loss_ref,
              m_ref, s_ref, ll_ref):
    j = pl.program_id(0)
    i = pl.program_id(1)
    rows = pl.ds(i * BM, BM)
    xb = x_ref[rows, :].astype(BF)
    wb = w_ref[...].astype(BF)
    lg = jnp.dot(xb, wb, preferred_element_type=jnp.float32)
    logit_ref[0] = lg
    col = jax.lax.broadcasted_iota(jnp.int32, (BM, BV), 1) + j * BV
    valid = col < V
    lgm = jnp.where(valid, lg, NEG)
    bm = jnp.max(lgm, axis=1, keepdims=True)
    lab = lab_ref[...]
    hit = jnp.sum(jnp.where(col == lab, lg, 0.0), axis=1, keepdims=True)

    @pl.when(j == 0)
    def _():
        m_ref[rows, :] = bm
        s_ref[rows, :] = jnp.sum(jnp.where(valid, jnp.exp(lgm - bm), 0.0),
                                 axis=1, keepdims=True)
        ll_ref[rows, :] = hit

    @pl.when(j > 0)
    def _():
        m_old = m_ref[rows, :]
        s_old = s_ref[rows, :]
        m_new = jnp.maximum(m_old, bm)
        p = jnp.sum(jnp.where(valid, jnp.exp(lgm - m_new), 0.0),
                    axis=1, keepdims=True)
        m_ref[rows, :] = m_new
        s_ref[rows, :] = s_old * jnp.exp(m_old - m_new) + p
        ll_ref[rows, :] = ll_ref[rows, :] + hit

    @pl.when(j == NJ - 1)
    def _():
        logz = m_ref[rows, :] + jnp.log(s_ref[rows, :])
        nll = logz - ll_ref[rows, :]
        ridx = jax.lax.broadcasted_iota(jnp.int32, (BM, 1), 0) + i * BM
        contrib = jnp.sum(jnp.where(ridx < S - 1, nll, 0.0)) / (S - 1.0)

        @pl.when(i == 0)
        def _():
            loss_ref[...] = jnp.full((1, 1), 0.0, jnp.float32) + contrib

        @pl.when(i > 0)
        def _():
            loss_ref[...] = loss_ref[...] + contrib


def _decode(x, w_dec, labels):
    return pl.pallas_call(
        _dec_body,
        grid=(NJ, S // BM),
        in_specs=[
            pl.BlockSpec((S, D), lambda j, i: (0, 0)),
            pl.BlockSpec((D, BV), lambda j, i: (0, j)),
            pl.BlockSpec((BM, 1), lambda j, i: (i, 0)),
        ],
        out_specs=[
            pl.BlockSpec((1, BM, BV), lambda j, i: (0, i, j)),
            pl.BlockSpec((1, 1), lambda j, i: (0, 0)),
        ],
        out_shape=[
            jax.ShapeDtypeStruct((B, S, V), jnp.float32),
            jax.ShapeDtypeStruct((1, 1), jnp.float32),
        ],
        scratch_shapes=[
            pltpu.VMEM((S, 1), jnp.float32),
            pltpu.VMEM((S, 1), jnp.float32),
            pltpu.VMEM((S, 1), jnp.float32),
        ],
        compiler_params=_arb(2),
    )(x, w_dec, labels)


def kernel(inputs, active, responses, peer_weights, Wqkv, bqkv, Wo, bo,
           W1, b1, W2, b2, ln1_g, ln1_b, ln2_g, ln2_b, W_dec):
    pw = peer_weights.reshape(1, NPEERS)
    af = active.astype(jnp.float32).reshape(1, NPEERS)
    nz = jax.random.normal(jax.random.key(42), (NPEERS,),
                           dtype=jnp.float32).reshape(1, NPEERS)
    resp = responses.reshape(TOPK, S, D)

    x = _combine(pw, af, nz, resp)

    bqkv3 = bqkv.reshape(L, 1, 3 * D)
    bo3 = bo.reshape(L, 1, D)
    b13 = b1.reshape(L, 1, FF)
    b23 = b2.reshape(L, 1, D)
    g13 = ln1_g.reshape(L, 1, D)
    bb13 = ln1_b.reshape(L, 1, D)
    g23 = ln2_g.reshape(L, 1, D)
    bb23 = ln2_b.reshape(L, 1, D)

    for l in range(L):
        qkv = _qkv(x, Wqkv, bqkv3, l)
        o = _attn(qkv)
        x = _oproj_ln(o, Wo, bo3, x, g13, bb13, l)
        x = _ffn_ln(x, W1, b13, W2, b23, g23, bb23, l)

    labels = jnp.concatenate(
        [inputs[0, 1:].astype(jnp.int32), jnp.zeros((1,), jnp.int32)]
    ).reshape(S, 1)

    logits, loss = _decode(x, W_dec, labels)

    return loss.reshape(()), logits


# speedup vs baseline: 1.3525x; 1.3525x over previous
import jax
import jax.numpy as jnp
from jax import lax
from jax.experimental import pallas as pl
from jax.experimental.pallas import tpu as pltpu

B, S, D, V = 1, 2048, 1024, 50258
L, H, FF = 2, 16, 4096
NPEERS, TOPK = 64, 8
HD = D // H

BM = 256
BV = 2048
NJ = (V + BV - 1) // BV
NEG = -1e30
BF = jnp.bfloat16

_arb = lambda n: pltpu.CompilerParams(dimension_semantics=("arbitrary",) * n)


def _combine_body(pw_ref, af_ref, nz_ref, resp_ref, x_ref):
    pw = pw_ref[...]
    af = af_ref[...]
    nz = nz_ref[...]
    n = jnp.sum(af)
    mean = jnp.sum(pw * af) / n
    std = jnp.sqrt(jnp.sum(af * (pw - mean) ** 2) / jnp.maximum(n - 1.0, 1.0))
    scores = jnp.where(af > 0, pw + nz * (std + 1e-7), -1e9)
    ms = []
    s = scores
    for _ in range(TOPK):
        m = jnp.max(s)
        ms.append(m)
        s = jnp.where(s >= m, NEG, s)
    ps = [jnp.exp(m - ms[0]) for m in ms]
    z = ps[0]
    for p in ps[1:]:
        z = z + p
    acc = (ps[0] / z) * resp_ref[0]
    for e in range(1, TOPK):
        acc = acc + (ps[e] / z) * resp_ref[e]
    x_ref[...] = acc


def _combine(pw, af, nz, resp):
    return pl.pallas_call(
        _combine_body,
        grid=(S // BM,),
        in_specs=[
            pl.BlockSpec((1, NPEERS), lambda i: (0, 0)),
            pl.BlockSpec((1, NPEERS), lambda i: (0, 0)),
            pl.BlockSpec((1, NPEERS), lambda i: (0, 0)),
            pl.BlockSpec((TOPK, BM, D), lambda i: (0, i, 0)),
        ],
        out_specs=pl.BlockSpec((BM, D), lambda i: (i, 0)),
        out_shape=jax.ShapeDtypeStruct((S, D), jnp.float32),
        compiler_params=_arb(1),
    )(pw, af, nz, resp)


def _qkv_body(x_ref, w_ref, b_ref, o_ref):
    xb = x_ref[...].astype(BF)
    wb = w_ref[0].astype(BF)
    o_ref[...] = (
        jnp.dot(xb, wb, preferred_element_type=jnp.float32) + b_ref[0]
    )


def _qkv(x, wqkv, bqkv, l):
    return pl.pallas_call(
        _qkv_body,
        grid=(S // BM, 3),
        in_specs=[
            pl.BlockSpec((BM, D), lambda i, j: (i, 0)),
            pl.BlockSpec((1, D, D), lambda i, j: (l, 0, j)),
            pl.BlockSpec((1, 1, D), lambda i, j: (l, 0, j)),
        ],
        out_specs=pl.BlockSpec((BM, D), lambda i, j: (i, j)),
        out_shape=jax.ShapeDtypeStruct((S, 3 * D), jnp.float32),
        compiler_params=_arb(2),
    )(x, wqkv, bqkv)


def _attn_one(q, k, v):
    s = lax.dot_general(q.astype(BF), k.astype(BF), (((1,), (1,)), ((), ())),
                        preferred_element_type=jnp.float32)
    s = s * (1.0 / (HD ** 0.5))
    m = jnp.max(s, axis=1, keepdims=True)
    p = jnp.exp(s - m)
    l = jnp.sum(p, axis=1, keepdims=True)
    att = (p / l).astype(BF)
    return jnp.dot(att, v.astype(BF), preferred_element_type=jnp.float32)


def _attn_body(q_ref, k_ref, v_ref, o_ref):
    q = q_ref[...]
    k = k_ref[...]
    v = v_ref[...]
    o0 = _attn_one(q[:, :HD], k[:, :HD], v[:, :HD])
    o1 = _attn_one(q[:, HD:], k[:, HD:], v[:, HD:])
    o_ref[...] = jnp.concatenate([o0, o1], axis=1)


def _attn(qkv):
    hp = H // 2
    return pl.pallas_call(
        _attn_body,
        grid=(hp, S // BM),
        in_specs=[
            pl.BlockSpec((BM, 2 * HD), lambda h, i: (i, h)),
            pl.BlockSpec((S, 2 * HD), lambda h, i: (0, hp + h)),
            pl.BlockSpec((S, 2 * HD), lambda h, i: (0, 2 * hp + h)),
        ],
        out_specs=pl.BlockSpec((BM, 2 * HD), lambda h, i: (i, h)),
        out_shape=jax.ShapeDtypeStruct((S, D), jnp.float32),
        compiler_params=_arb(2),
    )(qkv, qkv, qkv)


def _layernorm(y, g, b):
    mu = jnp.mean(y, axis=1, keepdims=True)
    var = jnp.mean((y - mu) ** 2, axis=1, keepdims=True)
    return (y - mu) * lax.rsqrt(var + 1e-5) * g + b


def _oproj_body(o_ref, w_ref, b_ref, x_ref, g_ref, bb_ref, y_ref):
    y = x_ref[...] + jnp.dot(o_ref[...].astype(BF), w_ref[0].astype(BF),
                             preferred_element_type=jnp.float32) + b_ref[0]
    y_ref[...] = _layernorm(y, g_ref[0], bb_ref[0])


def _oproj_ln(o, wo, bo, x, g, bb, l):
    return pl.pallas_call(
        _oproj_body,
        grid=(S // BM,),
        in_specs=[
            pl.BlockSpec((BM, D), lambda i: (i, 0)),
            pl.BlockSpec((1, D, D), lambda i: (l, 0, 0)),
            pl.BlockSpec((1, 1, D), lambda i: (l, 0, 0)),
            pl.BlockSpec((BM, D), lambda i: (i, 0)),
            pl.BlockSpec((1, 1, D), lambda i: (l, 0, 0)),
            pl.BlockSpec((1, 1, D), lambda i: (l, 0, 0)),
        ],
        out_specs=pl.BlockSpec((BM, D), lambda i: (i, 0)),
        out_shape=jax.ShapeDtypeStruct((S, D), jnp.float32),
        compiler_params=_arb(1),
    )(o, wo, bo, x, g, bb)


FC = 1024


def _ffn_body(x_ref, w1_ref, b1_ref, w2_ref, b2_ref, g_ref, bb_ref, y_ref,
              acc_ref):
    c = pl.program_id(1)
    h = jnp.maximum(
        jnp.dot(x_ref[...].astype(BF), w1_ref[0].astype(BF),
                preferred_element_type=jnp.float32) + b1_ref[0], 0.0)
    part = jnp.dot(h.astype(BF), w2_ref[0].astype(BF),
                   preferred_element_type=jnp.float32)

    @pl.when(c == 0)
    def _():
        acc_ref[...] = x_ref[...] + b2_ref[0] + part

    @pl.when(c > 0)
    def _():
        acc_ref[...] = acc_ref[...] + part

    @pl.when(c == FF // FC - 1)
    def _():
        y_ref[...] = _layernorm(acc_ref[...], g_ref[0], bb_ref[0])


def _ffn_ln(x, w1, b1, w2, b2, g, bb, l):
    return pl.pallas_call(
        _ffn_body,
        grid=(S // BM, FF // FC),
        in_specs=[
            pl.BlockSpec((BM, D), lambda i, c: (i, 0)),
            pl.BlockSpec((1, D, FC), lambda i, c: (l, 0, c)),
            pl.BlockSpec((1, 1, FC), lambda i, c: (l, 0, c)),
            pl.BlockSpec((1, FC, D), lambda i, c: (l, c, 0)),
            pl.BlockSpec((1, 1, D), lambda i, c: (l, 0, 0)),
            pl.BlockSpec((1, 1, D), lambda i, c: (l, 0, 0)),
            pl.BlockSpec((1, 1, D), lambda i, c: (l, 0, 0)),
        ],
        out_specs=pl.BlockSpec((BM, D), lambda i, c: (i, 0)),
        out_shape=jax.ShapeDtypeStruct((S, D), jnp.float32),
        scratch_shapes=[pltpu.VMEM((BM, D), jnp.float32)],
        compiler_params=_arb(2),
    )(x, w1, b1, w2, b2, g, bb)


def _dec_body(x_ref, w_ref, lab_ref, logit_ref, loss_ref,
              m_ref, s_ref, ll_ref):
    j = pl.program_id(0)
    i = pl.program_id(1)
    rows = pl.ds(i * BM, BM)
    xb = x_ref[rows, :].astype(BF)
    wb = w_ref[...].astype(BF)
    lg = jnp.dot(xb, wb, preferred_element_type=jnp.float32)
    logit_ref[...] = lg
    col = jax.lax.broadcasted_iota(jnp.int32, (BM, BV), 1) + j * BV
    valid = col < V
    lgm = jnp.where(valid, lg, NEG)
    bm = jnp.max(lgm, axis=1, keepdims=True)
    lab = lab_ref[...]
    hit = jnp.sum(jnp.where(col == lab, lg, 0.0), axis=1, keepdims=True)

    @pl.when(j == 0)
    def _():
        m_ref[rows, :] = bm
        s_ref[rows, :] = jnp.sum(jnp.where(valid, jnp.exp(lgm - bm), 0.0),
                                 axis=1, keepdims=True)
        ll_ref[rows, :] = hit

    @pl.when(j > 0)
    def _():
        m_old = m_ref[rows, :]
        s_old = s_ref[rows, :]
        m_new = jnp.maximum(m_old, bm)
        p = jnp.sum(jnp.where(valid, jnp.exp(lgm - m_new), 0.0),
                    axis=1, keepdims=True)
        m_ref[rows, :] = m_new
        s_ref[rows, :] = s_old * jnp.exp(m_old - m_new) + p
        ll_ref[rows, :] = ll_ref[rows, :] + hit

    @pl.when(j == NJ - 1)
    def _():
        logz = m_ref[rows, :] + jnp.log(s_ref[rows, :])
        nll = logz - ll_ref[rows, :]
        ridx = jax.lax.broadcasted_iota(jnp.int32, (BM, 1), 0) + i * BM
        contrib = jnp.sum(jnp.where(ridx < S - 1, nll, 0.0)) / (S - 1.0)

        @pl.when(i == 0)
        def _():
            loss_ref[...] = jnp.full((1, 1), 0.0, jnp.float32) + contrib

        @pl.when(i > 0)
        def _():
            loss_ref[...] = loss_ref[...] + contrib


def _decode(x, w_dec, labels):
    return pl.pallas_call(
        _dec_body,
        grid=(NJ, S // BM),
        in_specs=[
            pl.BlockSpec((S, D), lambda j, i: (0, 0)),
            pl.BlockSpec((D, BV), lambda j, i: (0, j)),
            pl.BlockSpec((BM, 1), lambda j, i: (i, 0)),
        ],
        out_specs=[
            pl.BlockSpec((BM, BV), lambda j, i: (i, j)),
            pl.BlockSpec((1, 1), lambda j, i: (0, 0)),
        ],
        out_shape=[
            jax.ShapeDtypeStruct((S, V), jnp.float32),
            jax.ShapeDtypeStruct((1, 1), jnp.float32),
        ],
        scratch_shapes=[
            pltpu.VMEM((S, 1), jnp.float32),
            pltpu.VMEM((S, 1), jnp.float32),
            pltpu.VMEM((S, 1), jnp.float32),
        ],
        compiler_params=_arb(2),
    )(x, w_dec, labels)


def kernel(inputs, active, responses, peer_weights, Wqkv, bqkv, Wo, bo,
           W1, b1, W2, b2, ln1_g, ln1_b, ln2_g, ln2_b, W_dec):
    pw = peer_weights.reshape(1, NPEERS)
    af = active.astype(jnp.float32).reshape(1, NPEERS)
    nz = jax.random.normal(jax.random.key(42), (NPEERS,),
                           dtype=jnp.float32).reshape(1, NPEERS)
    resp = responses.reshape(TOPK, S, D)

    x = _combine(pw, af, nz, resp)

    bqkv3 = bqkv.reshape(L, 1, 3 * D)
    bo3 = bo.reshape(L, 1, D)
    b13 = b1.reshape(L, 1, FF)
    b23 = b2.reshape(L, 1, D)
    g13 = ln1_g.reshape(L, 1, D)
    bb13 = ln1_b.reshape(L, 1, D)
    g23 = ln2_g.reshape(L, 1, D)
    bb23 = ln2_b.reshape(L, 1, D)

    for l in range(L):
        qkv = _qkv(x, Wqkv, bqkv3, l)
        o = _attn(qkv)
        x = _oproj_ln(o, Wo, bo3, x, g13, bb13, l)
        x = _ffn_ln(x, W1, b13, W2, b23, g23, bb23, l)

    labels = jnp.concatenate(
        [inputs[0, 1:].astype(jnp.int32), jnp.zeros((1,), jnp.int32)]
    ).reshape(S, 1)

    logits, loss = _decode(x, W_dec, labels)

    return loss.reshape(()), logits.reshape(B, S, V)


# speedup vs baseline: 1.3557x; 1.0023x over previous
import functools

import jax
import jax.numpy as jnp
from jax import lax
from jax.experimental import pallas as pl
from jax.experimental.pallas import tpu as pltpu
from jax.experimental.pallas import tpu_sc as plsc

B, S, D, V = 1, 2048, 1024, 50258
L, H, FF = 2, 16, 4096
NPEERS, TOPK = 64, 8
HD = D // H

BM = 256
BV = 2048
NJ = (V + BV - 1) // BV
NEG = -1e30
BF = jnp.bfloat16

_arb = lambda n: pltpu.CompilerParams(dimension_semantics=("arbitrary",) * n)


NC4 = NPEERS // 16


def _route_sc(pw, af, nz):
    mesh = plsc.VectorSubcoreMesh(core_axis_name="c", subcore_axis_name="s")

    @functools.partial(
        pl.kernel,
        out_type=jax.ShapeDtypeStruct((16,), jnp.float32),
        mesh=mesh,
        scratch_types=[
            pltpu.VMEM((NPEERS,), jnp.float32),
            pltpu.VMEM((NPEERS,), jnp.float32),
            pltpu.VMEM((NPEERS,), jnp.float32),
            pltpu.VMEM((16,), jnp.float32),
        ],
    )
    def k(pw_hbm, af_hbm, nz_hbm, out_hbm, pw_v, af_v, nz_v, w_v):
        wid = lax.axis_index("s") * 2 + lax.axis_index("c")
        lane = lax.iota(jnp.int32, 16)
        dnums = lax.GatherDimensionNumbers(
            offset_dims=(), collapsed_slice_dims=(0,), start_index_map=(0,))

        def shuffle(v, sh):
            idx = jnp.bitwise_xor(lane, sh).reshape(16, 1)
            return lax.gather(v, idx, dnums, (1,),
                              mode=lax.GatherScatterMode.PROMISE_IN_BOUNDS)

        def vsum(v):
            for sh in (1, 2, 4, 8):
                v = v + shuffle(v, sh)
            return v

        def vmax(v):
            for sh in (1, 2, 4, 8):
                v = jnp.maximum(v, shuffle(v, sh))
            return v

        @pl.when(wid == 0)
        def _():
            pltpu.sync_copy(pw_hbm, pw_v)
            pltpu.sync_copy(af_hbm, af_v)
            pltpu.sync_copy(nz_hbm, nz_v)
            pwc = [pw_v[pl.ds(c * 16, 16)] for c in range(NC4)]
            afc = [af_v[pl.ds(c * 16, 16)] for c in range(NC4)]
            nzc = [nz_v[pl.ds(c * 16, 16)] for c in range(NC4)]
            n = vsum(afc[0])
            for c in range(1, NC4):
                n = n + vsum(afc[c])
            tot = vsum(pwc[0] * afc[0])
            for c in range(1, NC4):
                tot = tot + vsum(pwc[c] * afc[c])
            mean = tot / n
            var = vsum(afc[0] * (pwc[0] - mean) * (pwc[0] - mean))
            for c in range(1, NC4):
                var = var + vsum(afc[c] * (pwc[c] - mean) * (pwc[c] - mean))
            var = var / jnp.maximum(n - 1.0, 1.0)
            y = 0.5 * (1.0 + var)
            for _ in range(14):
                y = 0.5 * (y + var / (y + 1e-30))
            std = y
            sc = [jnp.where(afc[c] > 0.0,
                            pwc[c] + nzc[c] * (std + 1e-7), -1e9)
                  for c in range(NC4)]
            mvec = jnp.where(lane < 0, 0.0, 0.0)
            ms0 = None
            for e in range(TOPK):
                m = vmax(sc[0])
                for c in range(1, NC4):
                    m = jnp.maximum(m, vmax(sc[c]))
                if ms0 is None:
                    ms0 = m
                mvec = jnp.where(lane == e, m, mvec)
                sc = [jnp.where(s >= m, NEG, s) for s in sc]
            pv = jnp.where(lane < TOPK, jnp.exp(mvec - ms0), 0.0)
            z = vsum(pv)
            w_v[...] = pv / z
            pltpu.sync_copy(w_v, out_hbm)

    return k(pw, af, nz)


def _combine_body(w_ref, resp_ref, x_ref):
    acc = w_ref[0] * resp_ref[0]
    for e in range(1, TOPK):
        acc = acc + w_ref[e] * resp_ref[e]
    x_ref[...] = acc


def _combine(w16, resp):
    return pl.pallas_call(
        _combine_body,
        grid=(S // BM,),
        in_specs=[
            pl.BlockSpec(memory_space=pltpu.SMEM),
            pl.BlockSpec((TOPK, BM, D), lambda i: (0, i, 0)),
        ],
        out_specs=pl.BlockSpec((BM, D), lambda i: (i, 0)),
        out_shape=jax.ShapeDtypeStruct((S, D), jnp.float32),
        compiler_params=_arb(1),
    )(w16, resp)


def _qkv_body(x_ref, w_ref, b_ref, o_ref):
    xb = x_ref[...].astype(BF)
    wb = w_ref[0].astype(BF)
    o_ref[...] = (
        jnp.dot(xb, wb, preferred_element_type=jnp.float32) + b_ref[0]
    )


def _qkv(x, wqkv, bqkv, l):
    return pl.pallas_call(
        _qkv_body,
        grid=(S // BM, 3),
        in_specs=[
            pl.BlockSpec((BM, D), lambda i, j: (i, 0)),
            pl.BlockSpec((1, D, D), lambda i, j: (l, 0, j)),
            pl.BlockSpec((1, 1, D), lambda i, j: (l, 0, j)),
        ],
        out_specs=pl.BlockSpec((BM, D), lambda i, j: (i, j)),
        out_shape=jax.ShapeDtypeStruct((S, 3 * D), jnp.float32),
        compiler_params=_arb(2),
    )(x, wqkv, bqkv)


def _attn_one(q, k, v):
    s = lax.dot_general(q.astype(BF), k.astype(BF), (((1,), (1,)), ((), ())),
                        preferred_element_type=jnp.float32)
    s = s * (1.0 / (HD ** 0.5))
    m = jnp.max(s, axis=1, keepdims=True)
    p = jnp.exp(s - m)
    l = jnp.sum(p, axis=1, keepdims=True)
    att = (p / l).astype(BF)
    return jnp.dot(att, v.astype(BF), preferred_element_type=jnp.float32)


def _attn_body(q_ref, k_ref, v_ref, o_ref):
    q = q_ref[...]
    k = k_ref[...]
    v = v_ref[...]
    o0 = _attn_one(q[:, :HD], k[:, :HD], v[:, :HD])
    o1 = _attn_one(q[:, HD:], k[:, HD:], v[:, HD:])
    o_ref[...] = jnp.concatenate([o0, o1], axis=1)


def _attn(qkv):
    hp = H // 2
    return pl.pallas_call(
        _attn_body,
        grid=(hp, S // BM),
        in_specs=[
            pl.BlockSpec((BM, 2 * HD), lambda h, i: (i, h)),
            pl.BlockSpec((S, 2 * HD), lambda h, i: (0, hp + h)),
            pl.BlockSpec((S, 2 * HD), lambda h, i: (0, 2 * hp + h)),
        ],
        out_specs=pl.BlockSpec((BM, 2 * HD), lambda h, i: (i, h)),
        out_shape=jax.ShapeDtypeStruct((S, D), jnp.float32),
        compiler_params=_arb(2),
    )(qkv, qkv, qkv)


def _layernorm(y, g, b):
    mu = jnp.mean(y, axis=1, keepdims=True)
    var = jnp.mean((y - mu) ** 2, axis=1, keepdims=True)
    return (y - mu) * lax.rsqrt(var + 1e-5) * g + b


def _oproj_body(o_ref, w_ref, b_ref, x_ref, g_ref, bb_ref, y_ref):
    y = x_ref[...] + jnp.dot(o_ref[...].astype(BF), w_ref[0].astype(BF),
                             preferred_element_type=jnp.float32) + b_ref[0]
    y_ref[...] = _layernorm(y, g_ref[0], bb_ref[0])


def _oproj_ln(o, wo, bo, x, g, bb, l):
    return pl.pallas_call(
        _oproj_body,
        grid=(S // BM,),
        in_specs=[
            pl.BlockSpec((BM, D), lambda i: (i, 0)),
            pl.BlockSpec((1, D, D), lambda i: (l, 0, 0)),
            pl.BlockSpec((1, 1, D), lambda i: (l, 0, 0)),
            pl.BlockSpec((BM, D), lambda i: (i, 0)),
            pl.BlockSpec((1, 1, D), lambda i: (l, 0, 0)),
            pl.BlockSpec((1, 1, D), lambda i: (l, 0, 0)),
        ],
        out_specs=pl.BlockSpec((BM, D), lambda i: (i, 0)),
        out_shape=jax.ShapeDtypeStruct((S, D), jnp.float32),
        compiler_params=_arb(1),
    )(o, wo, bo, x, g, bb)


FC = 1024


def _ffn_body(x_ref, w1_ref, b1_ref, w2_ref, b2_ref, g_ref, bb_ref, y_ref,
              acc_ref):
    c = pl.program_id(1)
    h = jnp.maximum(
        jnp.dot(x_ref[...].astype(BF), w1_ref[0].astype(BF),
                preferred_element_type=jnp.float32) + b1_ref[0], 0.0)
    part = jnp.dot(h.astype(BF), w2_ref[0].astype(BF),
                   preferred_element_type=jnp.float32)

    @pl.when(c == 0)
    def _():
        acc_ref[...] = x_ref[...] + b2_ref[0] + part

    @pl.when(c > 0)
    def _():
        acc_ref[...] = acc_ref[...] + part

    @pl.when(c == FF // FC - 1)
    def _():
        y_ref[...] = _layernorm(acc_ref[...], g_ref[0], bb_ref[0])


def _ffn_ln(x, w1, b1, w2, b2, g, bb, l):
    return pl.pallas_call(
        _ffn_body,
        grid=(S // BM, FF // FC),
        in_specs=[
            pl.BlockSpec((BM, D), lambda i, c: (i, 0)),
            pl.BlockSpec((1, D, FC), lambda i, c: (l, 0, c)),
            pl.BlockSpec((1, 1, FC), lambda i, c: (l, 0, c)),
            pl.BlockSpec((1, FC, D), lambda i, c: (l, c, 0)),
            pl.BlockSpec((1, 1, D), lambda i, c: (l, 0, 0)),
            pl.BlockSpec((1, 1, D), lambda i, c: (l, 0, 0)),
            pl.BlockSpec((1, 1, D), lambda i, c: (l, 0, 0)),
        ],
        out_specs=pl.BlockSpec((BM, D), lambda i, c: (i, 0)),
        out_shape=jax.ShapeDtypeStruct((S, D), jnp.float32),
        scratch_shapes=[pltpu.VMEM((BM, D), jnp.float32)],
        compiler_params=_arb(2),
    )(x, w1, b1, w2, b2, g, bb)


def _dec_body(x_ref, w_ref, lab_ref, logit_ref, loss_ref,
              m_ref, s_ref, ll_ref):
    j = pl.program_id(0)
    i = pl.program_id(1)
    rows = pl.ds(i * BM, BM)
    xb = x_ref[rows, :].astype(BF)
    wb = w_ref[...].astype(BF)
    lg = jnp.dot(xb, wb, preferred_element_type=jnp.float32)
    logit_ref[...] = lg
    col = jax.lax.broadcasted_iota(jnp.int32, (BM, BV), 1) + j * BV
    valid = col < V
    lgm = jnp.where(valid, lg, NEG)
    bm = jnp.max(lgm, axis=1, keepdims=True)
    lab = lab_ref[...]
    hit = jnp.sum(jnp.where(col == lab, lg, 0.0), axis=1, keepdims=True)

    @pl.when(j == 0)
    def _():
        m_ref[rows, :] = bm
        s_ref[rows, :] = jnp.sum(jnp.where(valid, jnp.exp(lgm - bm), 0.0),
                                 axis=1, keepdims=True)
        ll_ref[rows, :] = hit

    @pl.when(j > 0)
    def _():
        m_old = m_ref[rows, :]
        s_old = s_ref[rows, :]
        m_new = jnp.maximum(m_old, bm)
        p = jnp.sum(jnp.where(valid, jnp.exp(lgm - m_new), 0.0),
                    axis=1, keepdims=True)
        m_ref[rows, :] = m_new
        s_ref[rows, :] = s_old * jnp.exp(m_old - m_new) + p
        ll_ref[rows, :] = ll_ref[rows, :] + hit

    @pl.when(j == NJ - 1)
    def _():
        logz = m_ref[rows, :] + jnp.log(s_ref[rows, :])
        nll = logz - ll_ref[rows, :]
        ridx = jax.lax.broadcasted_iota(jnp.int32, (BM, 1), 0) + i * BM
        contrib = jnp.sum(jnp.where(ridx < S - 1, nll, 0.0)) / (S - 1.0)

        @pl.when(i == 0)
        def _():
            loss_ref[...] = jnp.full((1, 1), 0.0, jnp.float32) + contrib

        @pl.when(i > 0)
        def _():
            loss_ref[...] = loss_ref[...] + contrib


def _decode(x, w_dec, labels):
    return pl.pallas_call(
        _dec_body,
        grid=(NJ, S // BM),
        in_specs=[
            pl.BlockSpec((S, D), lambda j, i: (0, 0)),
            pl.BlockSpec((D, BV), lambda j, i: (0, j)),
            pl.BlockSpec((BM, 1), lambda j, i: (i, 0)),
        ],
        out_specs=[
            pl.BlockSpec((BM, BV), lambda j, i: (i, j)),
            pl.BlockSpec((1, 1), lambda j, i: (0, 0)),
        ],
        out_shape=[
            jax.ShapeDtypeStruct((S, V), jnp.float32),
            jax.ShapeDtypeStruct((1, 1), jnp.float32),
        ],
        scratch_shapes=[
            pltpu.VMEM((S, 1), jnp.float32),
            pltpu.VMEM((S, 1), jnp.float32),
            pltpu.VMEM((S, 1), jnp.float32),
        ],
        compiler_params=_arb(2),
    )(x, w_dec, labels)


def kernel(inputs, active, responses, peer_weights, Wqkv, bqkv, Wo, bo,
           W1, b1, W2, b2, ln1_g, ln1_b, ln2_g, ln2_b, W_dec):
    af = active.astype(jnp.float32)
    nz = jax.random.normal(jax.random.key(42), (NPEERS,), dtype=jnp.float32)
    resp = responses.reshape(TOPK, S, D)

    w16 = _route_sc(peer_weights, af, nz)
    x = _combine(w16, resp)

    bqkv3 = bqkv.reshape(L, 1, 3 * D)
    bo3 = bo.reshape(L, 1, D)
    b13 = b1.reshape(L, 1, FF)
    b23 = b2.reshape(L, 1, D)
    g13 = ln1_g.reshape(L, 1, D)
    bb13 = ln1_b.reshape(L, 1, D)
    g23 = ln2_g.reshape(L, 1, D)
    bb23 = ln2_b.reshape(L, 1, D)

    for l in range(L):
        qkv = _qkv(x, Wqkv, bqkv3, l)
        o = _attn(qkv)
        x = _oproj_ln(o, Wo, bo3, x, g13, bb13, l)
        x = _ffn_ln(x, W1, b13, W2, b23, g23, bb23, l)

    labels = jnp.concatenate(
        [inputs[0, 1:].astype(jnp.int32), jnp.zeros((1,), jnp.int32)]
    ).reshape(S, 1)

    logits, loss = _decode(x, W_dec, labels)

    return loss.reshape(()), logits.reshape(B, S, V)


# speedup vs baseline: 1.5270x; 1.1263x over previous
import functools

import jax
import jax.numpy as jnp
from jax import lax
from jax.experimental import pallas as pl
from jax.experimental.pallas import tpu as pltpu
from jax.experimental.pallas import tpu_sc as plsc

B, S, D, V = 1, 2048, 1024, 50258
L, H, FF = 2, 16, 4096
NPEERS, TOPK = 64, 8
HD = D // H

BM = 512
BV = 2048
NJ = (V + BV - 1) // BV
NEG = -1e30
BF = jnp.bfloat16

_arb = lambda n: pltpu.CompilerParams(dimension_semantics=("arbitrary",) * n)


NC4 = NPEERS // 16


def _route_sc(pw, af, nz):
    mesh = plsc.VectorSubcoreMesh(core_axis_name="c", subcore_axis_name="s")

    @functools.partial(
        pl.kernel,
        out_type=jax.ShapeDtypeStruct((16,), jnp.float32),
        mesh=mesh,
        scratch_types=[
            pltpu.VMEM((NPEERS,), jnp.float32),
            pltpu.VMEM((NPEERS,), jnp.float32),
            pltpu.VMEM((NPEERS,), jnp.float32),
            pltpu.VMEM((16,), jnp.float32),
        ],
    )
    def k(pw_hbm, af_hbm, nz_hbm, out_hbm, pw_v, af_v, nz_v, w_v):
        wid = lax.axis_index("s") * 2 + lax.axis_index("c")
        lane = lax.iota(jnp.int32, 16)
        dnums = lax.GatherDimensionNumbers(
            offset_dims=(), collapsed_slice_dims=(0,), start_index_map=(0,))

        def shuffle(v, sh):
            idx = jnp.bitwise_xor(lane, sh).reshape(16, 1)
            return lax.gather(v, idx, dnums, (1,),
                              mode=lax.GatherScatterMode.PROMISE_IN_BOUNDS)

        def vsum(v):
            for sh in (1, 2, 4, 8):
                v = v + shuffle(v, sh)
            return v

        def vmax(v):
            for sh in (1, 2, 4, 8):
                v = jnp.maximum(v, shuffle(v, sh))
            return v

        @pl.when(wid == 0)
        def _():
            pltpu.sync_copy(pw_hbm, pw_v)
            pltpu.sync_copy(af_hbm, af_v)
            pltpu.sync_copy(nz_hbm, nz_v)
            pwc = [pw_v[pl.ds(c * 16, 16)] for c in range(NC4)]
            afc = [af_v[pl.ds(c * 16, 16)] for c in range(NC4)]
            nzc = [nz_v[pl.ds(c * 16, 16)] for c in range(NC4)]
            n = vsum(afc[0])
            for c in range(1, NC4):
                n = n + vsum(afc[c])
            tot = vsum(pwc[0] * afc[0])
            for c in range(1, NC4):
                tot = tot + vsum(pwc[c] * afc[c])
            mean = tot / n
            var = vsum(afc[0] * (pwc[0] - mean) * (pwc[0] - mean))
            for c in range(1, NC4):
                var = var + vsum(afc[c] * (pwc[c] - mean) * (pwc[c] - mean))
            var = var / jnp.maximum(n - 1.0, 1.0)
            y = 0.5 * (1.0 + var)
            for _ in range(14):
                y = 0.5 * (y + var / (y + 1e-30))
            std = y
            sc = [jnp.where(afc[c] > 0.0,
                            pwc[c] + nzc[c] * (std + 1e-7), -1e9)
                  for c in range(NC4)]
            mvec = jnp.where(lane < 0, 0.0, 0.0)
            ms0 = None
            for e in range(TOPK):
                m = vmax(sc[0])
                for c in range(1, NC4):
                    m = jnp.maximum(m, vmax(sc[c]))
                if ms0 is None:
                    ms0 = m
                mvec = jnp.where(lane == e, m, mvec)
                sc = [jnp.where(s >= m, NEG, s) for s in sc]
            pv = jnp.where(lane < TOPK, jnp.exp(mvec - ms0), 0.0)
            z = vsum(pv)
            w_v[...] = pv / z
            pltpu.sync_copy(w_v, out_hbm)

    return k(pw, af, nz)


def _combine_body(w_ref, resp_ref, x_ref):
    acc = w_ref[0] * resp_ref[0]
    for e in range(1, TOPK):
        acc = acc + w_ref[e] * resp_ref[e]
    x_ref[...] = acc


def _combine(w16, resp):
    return pl.pallas_call(
        _combine_body,
        grid=(S // BM,),
        in_specs=[
            pl.BlockSpec(memory_space=pltpu.SMEM),
            pl.BlockSpec((TOPK, BM, D), lambda i: (0, i, 0)),
        ],
        out_specs=pl.BlockSpec((BM, D), lambda i: (i, 0)),
        out_shape=jax.ShapeDtypeStruct((S, D), jnp.float32),
        compiler_params=_arb(1),
    )(w16, resp)


def _qkv_body(x_ref, w_ref, b_ref, o_ref):
    xb = x_ref[...].astype(BF)
    wb = w_ref[0].astype(BF)
    o_ref[...] = (
        jnp.dot(xb, wb, preferred_element_type=jnp.float32) + b_ref[0]
    )


def _qkv(x, wqkv, bqkv, l):
    return pl.pallas_call(
        _qkv_body,
        grid=(S // BM, 3),
        in_specs=[
            pl.BlockSpec((BM, D), lambda i, j: (i, 0)),
            pl.BlockSpec((1, D, D), lambda i, j: (l, 0, j)),
            pl.BlockSpec((1, 1, D), lambda i, j: (l, 0, j)),
        ],
        out_specs=pl.BlockSpec((BM, D), lambda i, j: (i, j)),
        out_shape=jax.ShapeDtypeStruct((S, 3 * D), jnp.float32),
        compiler_params=_arb(2),
    )(x, wqkv, bqkv)


def _attn_one(q, k, v):
    s = lax.dot_general(q.astype(BF), k.astype(BF), (((1,), (1,)), ((), ())),
                        preferred_element_type=jnp.float32)
    s = s * (1.0 / (HD ** 0.5))
    m = jnp.max(s, axis=1, keepdims=True)
    p = jnp.exp(s - m)
    l = jnp.sum(p, axis=1, keepdims=True)
    att = (p / l).astype(BF)
    return jnp.dot(att, v.astype(BF), preferred_element_type=jnp.float32)


def _attn_body(q_ref, k_ref, v_ref, o_ref):
    q = q_ref[...]
    k = k_ref[...]
    v = v_ref[...]
    o0 = _attn_one(q[:, :HD], k[:, :HD], v[:, :HD])
    o1 = _attn_one(q[:, HD:], k[:, HD:], v[:, HD:])
    o_ref[...] = jnp.concatenate([o0, o1], axis=1)


def _attn(qkv):
    hp = H // 2
    return pl.pallas_call(
        _attn_body,
        grid=(hp, S // BM),
        in_specs=[
            pl.BlockSpec((BM, 2 * HD), lambda h, i: (i, h)),
            pl.BlockSpec((S, 2 * HD), lambda h, i: (0, hp + h)),
            pl.BlockSpec((S, 2 * HD), lambda h, i: (0, 2 * hp + h)),
        ],
        out_specs=pl.BlockSpec((BM, 2 * HD), lambda h, i: (i, h)),
        out_shape=jax.ShapeDtypeStruct((S, D), jnp.float32),
        compiler_params=_arb(2),
    )(qkv, qkv, qkv)


def _layernorm(y, g, b):
    mu = jnp.mean(y, axis=1, keepdims=True)
    var = jnp.mean((y - mu) ** 2, axis=1, keepdims=True)
    return (y - mu) * lax.rsqrt(var + 1e-5) * g + b


def _oproj_body(o_ref, w_ref, b_ref, x_ref, g_ref, bb_ref, y_ref):
    y = x_ref[...] + jnp.dot(o_ref[...].astype(BF), w_ref[0].astype(BF),
                             preferred_element_type=jnp.float32) + b_ref[0]
    y_ref[...] = _layernorm(y, g_ref[0], bb_ref[0])


def _oproj_ln(o, wo, bo, x, g, bb, l):
    return pl.pallas_call(
        _oproj_body,
        grid=(S // BM,),
        in_specs=[
            pl.BlockSpec((BM, D), lambda i: (i, 0)),
            pl.BlockSpec((1, D, D), lambda i: (l, 0, 0)),
            pl.BlockSpec((1, 1, D), lambda i: (l, 0, 0)),
            pl.BlockSpec((BM, D), lambda i: (i, 0)),
            pl.BlockSpec((1, 1, D), lambda i: (l, 0, 0)),
            pl.BlockSpec((1, 1, D), lambda i: (l, 0, 0)),
        ],
        out_specs=pl.BlockSpec((BM, D), lambda i: (i, 0)),
        out_shape=jax.ShapeDtypeStruct((S, D), jnp.float32),
        compiler_params=_arb(1),
    )(o, wo, bo, x, g, bb)


FC = 1024


def _ffn_body(x_ref, w1_ref, b1_ref, w2_ref, b2_ref, g_ref, bb_ref, y_ref,
              acc_ref):
    c = pl.program_id(1)
    h = jnp.maximum(
        jnp.dot(x_ref[...].astype(BF), w1_ref[0].astype(BF),
                preferred_element_type=jnp.float32) + b1_ref[0], 0.0)
    part = jnp.dot(h.astype(BF), w2_ref[0].astype(BF),
                   preferred_element_type=jnp.float32)

    @pl.when(c == 0)
    def _():
        acc_ref[...] = x_ref[...] + b2_ref[0] + part

    @pl.when(c > 0)
    def _():
        acc_ref[...] = acc_ref[...] + part

    @pl.when(c == FF // FC - 1)
    def _():
        y_ref[...] = _layernorm(acc_ref[...], g_ref[0], bb_ref[0])


def _ffn_ln(x, w1, b1, w2, b2, g, bb, l):
    return pl.pallas_call(
        _ffn_body,
        grid=(S // BM, FF // FC),
        in_specs=[
            pl.BlockSpec((BM, D), lambda i, c: (i, 0)),
            pl.BlockSpec((1, D, FC), lambda i, c: (l, 0, c)),
            pl.BlockSpec((1, 1, FC), lambda i, c: (l, 0, c)),
            pl.BlockSpec((1, FC, D), lambda i, c: (l, c, 0)),
            pl.BlockSpec((1, 1, D), lambda i, c: (l, 0, 0)),
            pl.BlockSpec((1, 1, D), lambda i, c: (l, 0, 0)),
            pl.BlockSpec((1, 1, D), lambda i, c: (l, 0, 0)),
        ],
        out_specs=pl.BlockSpec((BM, D), lambda i, c: (i, 0)),
        out_shape=jax.ShapeDtypeStruct((S, D), jnp.float32),
        scratch_shapes=[pltpu.VMEM((BM, D), jnp.float32)],
        compiler_params=_arb(2),
    )(x, w1, b1, w2, b2, g, bb)


def _dec_body(x_ref, w_ref, lab_ref, logit_ref, loss_ref,
              m_ref, s_ref, ll_ref):
    j = pl.program_id(0)
    i = pl.program_id(1)
    rows = pl.ds(i * BM, BM)
    xb = x_ref[rows, :].astype(BF)
    wb = w_ref[...].astype(BF)
    lg = jnp.dot(xb, wb, preferred_element_type=jnp.float32)
    logit_ref[...] = lg
    col = jax.lax.broadcasted_iota(jnp.int32, (BM, BV), 1) + j * BV
    valid = col < V
    lgm = jnp.where(valid, lg, NEG)
    bm = jnp.max(lgm, axis=1, keepdims=True)
    lab = lab_ref[...]
    hit = jnp.sum(jnp.where(col == lab, lg, 0.0), axis=1, keepdims=True)

    @pl.when(j == 0)
    def _():
        m_ref[rows, :] = bm
        s_ref[rows, :] = jnp.sum(jnp.where(valid, jnp.exp(lgm - bm), 0.0),
                                 axis=1, keepdims=True)
        ll_ref[rows, :] = hit

    @pl.when(j > 0)
    def _():
        m_old = m_ref[rows, :]
        s_old = s_ref[rows, :]
        m_new = jnp.maximum(m_old, bm)
        p = jnp.sum(jnp.where(valid, jnp.exp(lgm - m_new), 0.0),
                    axis=1, keepdims=True)
        m_ref[rows, :] = m_new
        s_ref[rows, :] = s_old * jnp.exp(m_old - m_new) + p
        ll_ref[rows, :] = ll_ref[rows, :] + hit

    @pl.when(j == NJ - 1)
    def _():
        logz = m_ref[rows, :] + jnp.log(s_ref[rows, :])
        nll = logz - ll_ref[rows, :]
        ridx = jax.lax.broadcasted_iota(jnp.int32, (BM, 1), 0) + i * BM
        contrib = jnp.sum(jnp.where(ridx < S - 1, nll, 0.0)) / (S - 1.0)

        @pl.when(i == 0)
        def _():
            loss_ref[...] = jnp.full((1, 1), 0.0, jnp.float32) + contrib

        @pl.when(i > 0)
        def _():
            loss_ref[...] = loss_ref[...] + contrib


def _decode(x, w_dec, labels):
    return pl.pallas_call(
        _dec_body,
        grid=(NJ, S // BM),
        in_specs=[
            pl.BlockSpec((S, D), lambda j, i: (0, 0)),
            pl.BlockSpec((D, BV), lambda j, i: (0, j)),
            pl.BlockSpec((BM, 1), lambda j, i: (i, 0)),
        ],
        out_specs=[
            pl.BlockSpec((BM, BV), lambda j, i: (i, j)),
            pl.BlockSpec((1, 1), lambda j, i: (0, 0)),
        ],
        out_shape=[
            jax.ShapeDtypeStruct((S, V), jnp.float32),
            jax.ShapeDtypeStruct((1, 1), jnp.float32),
        ],
        scratch_shapes=[
            pltpu.VMEM((S, 1), jnp.float32),
            pltpu.VMEM((S, 1), jnp.float32),
            pltpu.VMEM((S, 1), jnp.float32),
        ],
        compiler_params=_arb(2),
    )(x, w_dec, labels)


def kernel(inputs, active, responses, peer_weights, Wqkv, bqkv, Wo, bo,
           W1, b1, W2, b2, ln1_g, ln1_b, ln2_g, ln2_b, W_dec):
    af = active.astype(jnp.float32)
    nz = jax.random.normal(jax.random.key(42), (NPEERS,), dtype=jnp.float32)
    resp = responses.reshape(TOPK, S, D)

    w16 = _route_sc(peer_weights, af, nz)
    x = _combine(w16, resp)

    bqkv3 = bqkv.reshape(L, 1, 3 * D)
    bo3 = bo.reshape(L, 1, D)
    b13 = b1.reshape(L, 1, FF)
    b23 = b2.reshape(L, 1, D)
    g13 = ln1_g.reshape(L, 1, D)
    bb13 = ln1_b.reshape(L, 1, D)
    g23 = ln2_g.reshape(L, 1, D)
    bb23 = ln2_b.reshape(L, 1, D)

    for l in range(L):
        qkv = _qkv(x, Wqkv, bqkv3, l)
        o = _attn(qkv)
        x = _oproj_ln(o, Wo, bo3, x, g13, bb13, l)
        x = _ffn_ln(x, W1, b13, W2, b23, g23, bb23, l)

    labels = jnp.concatenate(
        [inputs[0, 1:].astype(jnp.int32), jnp.zeros((1,), jnp.int32)]
    ).reshape(S, 1)

    logits, loss = _decode(x, W_dec, labels)

    return loss.reshape(()), logits.reshape(B, S, V)
